# Initial kernel scaffold; baseline (speedup 1.0000x reference)
import jax
import jax.numpy as jnp
from jax.experimental import pallas as pl


def kernel(x, positions, next_positions, token_table, pos_table, npos_table):
    raise NotImplementedError("write your pallas kernel here")


# speedup vs baseline: 3.3109x; 3.3109x over previous
import functools

import jax
import jax.numpy as jnp
from jax import lax
from jax.experimental import pallas as pl
from jax.experimental.pallas import tpu as pltpu
from jax.experimental.pallas import tpu_sc as plsc

_NC = 2
_NS = 16
_NW = _NC * _NS
_IDXW = 128
_SUB = 4
_CHUNK = _IDXW * _SUB


@functools.partial(jax.jit, static_argnums=(6, 7))
def _embed_sum(x2d, p2d, n2d, tok, pos, npos, n_rows, d):
    n_per_w = n_rows // _NW
    n_chunks = n_per_w // _CHUNK
    units_per_w = n_per_w // _IDXW

    mesh = plsc.VectorSubcoreMesh(core_axis_name="c", subcore_axis_name="s",
                                  num_cores=_NC, num_subcores=_NS)
    nvec = d // 16

    @functools.partial(
        pl.kernel,
        out_type=jax.ShapeDtypeStruct((n_rows, d), jnp.float32),
        mesh=mesh,
        compiler_params=pltpu.CompilerParams(use_tc_tiling_on_sc=False),
        scratch_types=[
            pltpu.VMEM((_SUB, _IDXW), jnp.int32),
            pltpu.VMEM((_SUB, _IDXW), jnp.int32),
            pltpu.VMEM((_SUB, _IDXW), jnp.int32),
            pltpu.VMEM((_CHUNK, d), jnp.float32),
            pltpu.VMEM((_CHUNK, d), jnp.float32),
            pltpu.VMEM((_CHUNK, d), jnp.float32),
            pltpu.SemaphoreType.DMA,
        ],
    )
    def k(x_hbm, p_hbm, np_hbm, tok_hbm, pos_hbm, npos_hbm, out_hbm,
          xidx, pidx, nidx, acc, pbuf, nbuf, sem):
        wid = lax.axis_index("s") * _NC + lax.axis_index("c")
        u0 = wid * units_per_w
        r0 = wid * n_per_w

        def chunk_body(c, carry):
            u = u0 + c * _SUB
            rbase = r0 + c * _CHUNK
            pltpu.sync_copy(x_hbm.at[pl.ds(u, _SUB)], xidx)
            pltpu.sync_copy(p_hbm.at[pl.ds(u, _SUB)], pidx)
            pltpu.sync_copy(np_hbm.at[pl.ds(u, _SUB)], nidx)
            cps = []
            for kk in range(_SUB):
                dst = pl.ds(kk * _IDXW, _IDXW)
                cps.append(pltpu.async_copy(
                    tok_hbm.at[xidx.at[kk]], acc.at[dst], sem))
                cps.append(pltpu.async_copy(
                    pos_hbm.at[pidx.at[kk]], pbuf.at[dst], sem))
                cps.append(pltpu.async_copy(
                    npos_hbm.at[nidx.at[kk]], nbuf.at[dst], sem))
            for cp in cps:
                cp.wait()

            def add_body(i, carry2):
                for j in range(nvec):
                    sl = pl.ds(j * 16, 16)
                    acc[i, sl] = acc[i, sl] + pbuf[i, sl] + nbuf[i, sl]
                return carry2

            lax.fori_loop(0, _CHUNK, add_body, 0)
            pltpu.sync_copy(acc, out_hbm.at[pl.ds(rbase, _CHUNK)])
            return carry

        lax.fori_loop(0, n_chunks, chunk_body, 0)

    return k(x2d, p2d, n2d, tok, pos, npos)


def kernel(x, positions, next_positions, token_table, pos_table, npos_table):
    b, t = x.shape
    d = token_table.shape[1]
    n_rows = b * t
    x2d = x.reshape(n_rows // _IDXW, _IDXW).astype(jnp.int32)
    p2d = positions.reshape(n_rows // _IDXW, _IDXW).astype(jnp.int32)
    n2d = next_positions.reshape(n_rows // _IDXW, _IDXW).astype(jnp.int32)
    out = _embed_sum(x2d, p2d, n2d, token_table, pos_table, npos_table,
                     n_rows, d)
    return out.reshape(b, t, d)


# speedup vs baseline: 3.3230x; 1.0037x over previous
import functools

import jax
import jax.numpy as jnp
from jax import lax
from jax.experimental import pallas as pl
from jax.experimental.pallas import tpu as pltpu
from jax.experimental.pallas import tpu_sc as plsc

_NC = 2
_NS = 16
_NW = _NC * _NS
_CHUNK = 128


@functools.partial(jax.jit, static_argnums=(4, 5, 6))
def _embed_sum(idx_all, tok, pos, npos, n_rows, max_len, d):
    n_per_w = n_rows // _NW
    n_chunks = n_per_w // _CHUNK
    nvec = d // 16

    mesh = plsc.VectorSubcoreMesh(core_axis_name="c", subcore_axis_name="s",
                                  num_cores=_NC, num_subcores=_NS)

    @functools.partial(
        pl.kernel,
        out_type=jax.ShapeDtypeStruct((n_rows, d), jnp.float32),
        mesh=mesh,
        compiler_params=pltpu.CompilerParams(use_tc_tiling_on_sc=False),
        scratch_types=[
            pltpu.VMEM((2, 3, _CHUNK), jnp.int32),
            pltpu.VMEM((2, _CHUNK, d), jnp.float32),
            pltpu.VMEM((2, _CHUNK, d), jnp.float32),
            pltpu.VMEM((2, _CHUNK, d), jnp.float32),
            pltpu.SemaphoreType.DMA,
            pltpu.SemaphoreType.DMA,
            pltpu.SemaphoreType.DMA,
            pltpu.SemaphoreType.DMA,
        ],
    )
    def k(idx_hbm, tok_hbm, pos_hbm, npos_hbm, out_hbm,
          idx, acc, pbuf, nbuf, isem0, isem1, gsem0, gsem1):
        sid = lax.axis_index("s")
        wid = sid * _NC + lax.axis_index("c")
        u0 = wid * n_chunks
        r0 = wid * n_per_w
        isem = (isem0, isem1)
        gsem = (gsem0, gsem1)

        def g_issue(s):
            pltpu.async_copy(tok_hbm.at[idx.at[s, 0]], acc.at[s], gsem[s])
            pltpu.async_copy(pos_hbm.at[idx.at[s, 1]], pbuf.at[s], gsem[s])
            pltpu.async_copy(npos_hbm.at[idx.at[s, 2]], nbuf.at[s], gsem[s])

        def g_wait(s):
            pltpu.make_async_copy(tok_hbm.at[idx.at[s, 0]], acc.at[s],
                                  gsem[s]).wait()
            pltpu.make_async_copy(pos_hbm.at[idx.at[s, 1]], pbuf.at[s],
                                  gsem[s]).wait()
            pltpu.make_async_copy(npos_hbm.at[idx.at[s, 2]], nbuf.at[s],
                                  gsem[s]).wait()

        def i_wait(s):
            pltpu.make_async_copy(idx_hbm.at[u0], idx.at[s], isem[s]).wait()

        pltpu.async_copy(idx_hbm.at[u0], idx.at[0], isem0)
        i_wait(0)
        g_issue(0)
        pltpu.async_copy(idx_hbm.at[u0 + jnp.minimum(1, n_chunks - 1)],
                         idx.at[1], isem1)

        def half(c, s):
            so = 1 - s
            i_wait(so)
            g_issue(so)
            g_wait(s)
            u2 = u0 + jnp.minimum(c + 2, n_chunks - 1)
            pltpu.async_copy(idx_hbm.at[u2], idx.at[s], isem[s])

            def add_body(i, carry):
                for j in range(nvec):
                    sl = pl.ds(j * 16, 16)
                    acc[s, i, sl] = acc[s, i, sl] + pbuf[s, i, sl] + nbuf[s, i, sl]
                return carry

            lax.fori_loop(0, _CHUNK, add_body, 0)
            pltpu.sync_copy(acc.at[s], out_hbm.at[pl.ds(r0 + c * _CHUNK,
                                                        _CHUNK)])

        def body(i, carry):
            c = i * 2
            half(c, 0)
            half(c + 1, 1)
            return carry

        lax.fori_loop(0, n_chunks // 2, body, 0)
        i_wait(1)
        g_wait(0)

    return k(idx_all, tok, pos, npos)


def kernel(x, positions, next_positions, token_table, pos_table, npos_table):
    b, t = x.shape
    d = token_table.shape[1]
    max_len = pos_table.shape[0]
    n_rows = b * t
    units = n_rows // _CHUNK
    idx_all = jnp.stack(
        [x.reshape(units, _CHUNK).astype(jnp.int32),
         positions.reshape(units, _CHUNK).astype(jnp.int32),
         next_positions.reshape(units, _CHUNK).astype(jnp.int32)], axis=1)
    out = _embed_sum(idx_all, token_table, pos_table, npos_table,
                     n_rows, max_len, d)
    return out.reshape(b, t, d)
